# baseline (speedup 1.0000x reference)
import jax
import jax.numpy as jnp
from jax.experimental import pallas as pl

B, N, D, E = 4, 10000, 128, 160000
EC = E + 2 * N
CHUNK = 20000
NCH = EC // CHUNK


def _x_body(obs_ref, x_ref, bv_ref):
    o = obs_ref[0]
    x_ref[0, :N, :] = o
    x_ref[0, pl.ds(N, 1), :] = jnp.mean(o, axis=0, keepdims=True)
    bv_ref[...] = jnp.full((1, 1, N + 1), pl.program_id(0), jnp.int32)


def _edge_body(ei_ref, ea_ref, oi_ref, oa_ref):
    off = pl.program_id(0) * (N + 1)
    pos = jax.lax.broadcasted_iota(jnp.int32, (1, 2 * N), 1)
    r0 = jnp.where(pos < N, N, pos - N)
    r1 = jnp.where(pos < N, pos, N)
    virt = jnp.concatenate([r0, r1], axis=0)
    ei = ei_ref[:, 0, :]
    oi_ref[:, 0, 0, :] = jnp.concatenate([ei, virt], axis=1) + off
    oa_ref[0, 0, :] = jnp.concatenate(
        [ea_ref[0, 0, :], jnp.ones((2 * N,), jnp.float32)])


def kernel(obs, edge_index, edge_attr):
    idt = edge_index.dtype

    x_ext, bv = pl.pallas_call(
        _x_body,
        grid=(B,),
        in_specs=[pl.BlockSpec((1, N, D), lambda b: (b, 0, 0))],
        out_specs=[
            pl.BlockSpec((1, N + 1, D), lambda b: (b, 0, 0)),
            pl.BlockSpec((1, 1, N + 1), lambda b: (b, 0, 0)),
        ],
        out_shape=[
            jax.ShapeDtypeStruct((B, N + 1, D), obs.dtype),
            jax.ShapeDtypeStruct((B, 1, N + 1), jnp.int32),
        ],
    )(obs)

    ei3 = edge_index.reshape(2, 1, E)
    ea3 = edge_attr.reshape(1, 1, E)
    eindex4, eattr3 = pl.pallas_call(
        _edge_body,
        grid=(B,),
        in_specs=[
            pl.BlockSpec((2, 1, E), lambda b: (0, 0, 0)),
            pl.BlockSpec((1, 1, E), lambda b: (0, 0, 0)),
        ],
        out_specs=[
            pl.BlockSpec((2, 1, 1, EC), lambda b: (0, b, 0, 0)),
            pl.BlockSpec((1, 1, EC), lambda b: (b, 0, 0)),
        ],
        out_shape=[
            jax.ShapeDtypeStruct((2, B, 1, EC), idt),
            jax.ShapeDtypeStruct((B, 1, EC), edge_attr.dtype),
        ],
    )(ei3, ea3)
    eindex = eindex4.reshape(2, B * EC)

    x = x_ext.reshape(B * (N + 1), D)
    eattr = eattr3.reshape(B * EC, 1)
    batch_vec = bv.reshape(B * (N + 1))
    return x, eindex, eattr, batch_vec


# speedup vs baseline: 1.3845x; 1.3845x over previous
import functools

import jax
import jax.numpy as jnp
from jax import lax
from jax.experimental import pallas as pl
from jax.experimental.pallas import tpu as pltpu
from jax.experimental.pallas import tpu_sc as plsc

B, N, D, E = 4, 10000, 128, 160000
EC = E + 2 * N
CH = 10000
KPR = EC // CH
NROWS = 12
NJOBS = NROWS * KPR
NWORK = 32
VPC = CH // 16


def _x_body(obs_ref, x_ref, bv_ref):
    o = obs_ref[0]
    x_ref[0, :N, :] = o
    x_ref[0, pl.ds(N, 1), :] = jnp.mean(o, axis=0, keepdims=True)
    bv_ref[...] = jnp.full((1, 1, N + 1), pl.program_id(0), jnp.int32)


def _edge_sc_body(ei_hbm, ea_hbm, oi_hbm, oa_hbm, bufi, buff, ones_v):
    wid = lax.axis_index("c") * 16 + lax.axis_index("s")

    def fill_ones(i, c):
        ones_v[pl.ds(i * 16, 16)] = jnp.full((16,), 1.0, jnp.float32)
        return c
    lax.fori_loop(0, VPC, fill_ones, 0)

    def do_job(j):
        row = j // KPR
        k = j % KPR
        is_ei = row < 8
        r = row // 4
        bb = jnp.where(is_ei, row % 4, row - 8)
        off = bb * (N + 1)
        dst = jnp.where(is_ei, r * (B * EC) + bb * EC, bb * EC) + k * CH

        @pl.when(is_ei & (k < KPR - 2))
        def _copy_add():
            pltpu.sync_copy(ei_hbm.at[pl.ds(r * E + k * CH, CH)], bufi)

            def addl(i, c):
                sl = pl.ds(i * 16, 16)
                bufi[sl] = bufi[sl] + off
                return c
            lax.fori_loop(0, VPC, addl, 0)
            pltpu.sync_copy(bufi, oi_hbm.at[pl.ds(dst, CH)])

        @pl.when(is_ei & (k >= KPR - 2))
        def _gen_virtual():
            vbase = (k - (KPR - 2)) * CH

            def genl(i, c):
                pos = lax.iota(jnp.int32, 16) + (vbase + i * 16)
                v0 = jnp.where(pos < N, N, pos - N)
                v1 = jnp.where(pos < N, pos, N)
                bufi[pl.ds(i * 16, 16)] = jnp.where(r == 0, v0, v1) + off
                return c
            lax.fori_loop(0, VPC, genl, 0)
            pltpu.sync_copy(bufi, oi_hbm.at[pl.ds(dst, CH)])

        @pl.when(jnp.logical_not(is_ei) & (k < KPR - 2))
        def _attr_copy():
            pltpu.sync_copy(ea_hbm.at[pl.ds(k * CH, CH)], buff)
            pltpu.sync_copy(buff, oa_hbm.at[pl.ds(dst, CH)])

        @pl.when(jnp.logical_not(is_ei) & (k >= KPR - 2))
        def _attr_ones():
            pltpu.sync_copy(ones_v, oa_hbm.at[pl.ds(dst, CH)])

    def tloop(t, c):
        j = wid + NWORK * t

        @pl.when(j < NJOBS)
        def _():
            do_job(j)
        return c
    lax.fori_loop(0, (NJOBS + NWORK - 1) // NWORK, tloop, 0)


def kernel(obs, edge_index, edge_attr):
    idt = edge_index.dtype

    x_ext, bv = pl.pallas_call(
        _x_body,
        grid=(B,),
        in_specs=[pl.BlockSpec((1, N, D), lambda b: (b, 0, 0))],
        out_specs=[
            pl.BlockSpec((1, N + 1, D), lambda b: (b, 0, 0)),
            pl.BlockSpec((1, 1, N + 1), lambda b: (b, 0, 0)),
        ],
        out_shape=[
            jax.ShapeDtypeStruct((B, N + 1, D), obs.dtype),
            jax.ShapeDtypeStruct((B, 1, N + 1), jnp.int32),
        ],
    )(obs)

    edge_sc = functools.partial(
        pl.kernel,
        mesh=plsc.VectorSubcoreMesh(core_axis_name="c", subcore_axis_name="s"),
        out_type=[
            jax.ShapeDtypeStruct((2 * B * EC,), idt),
            jax.ShapeDtypeStruct((B * EC,), edge_attr.dtype),
        ],
        scratch_types=[
            pltpu.VMEM((CH,), jnp.int32),
            pltpu.VMEM((CH,), jnp.float32),
            pltpu.VMEM((CH,), jnp.float32),
        ],
    )(_edge_sc_body)
    eif, eaf = edge_sc(edge_index.reshape(2 * E), edge_attr.reshape(E))

    x = x_ext.reshape(B * (N + 1), D)
    eindex = eif.reshape(2, B * EC)
    eattr = eaf.reshape(B * EC, 1)
    batch_vec = bv.reshape(B * (N + 1))
    return x, eindex, eattr, batch_vec


# speedup vs baseline: 1.4292x; 1.0323x over previous
import functools

import jax
import jax.numpy as jnp
from jax import lax
from jax.experimental import pallas as pl
from jax.experimental.pallas import tpu as pltpu
from jax.experimental.pallas import tpu_sc as plsc

B, N, D, E = 4, 10000, 128, 160000
EC = E + 2 * N
NV = B * (N + 1)
CH = 10000
KPR = EC // CH
NROWS = 12
NJOBS = NROWS * KPR
NWORK = 32
VPC = CH // 16


def _x_body(obs_ref, x_ref):
    b = pl.program_id(0)
    o = obs_ref[0]
    m = jnp.mean(o, axis=0, keepdims=True)

    for bb in range(B):
        @pl.when(b == bb)
        def _(bb=bb):
            x_ref[pl.ds(bb * (N + 1), N), :] = o
            x_ref[pl.ds(bb * (N + 1) + N, 1), :] = m


def _edge_sc_body(ei_hbm, ea_hbm, ones_hbm, oi_hbm, oa_hbm, ob_hbm,
                  bufi, buff, onesv, bufbv):
    wid = lax.axis_index("c") * 16 + lax.axis_index("s")
    pltpu.sync_copy(ones_hbm, onesv)

    def do_job(j):
        row = j // KPR
        k = j % KPR
        is_ei = row < 8
        r = row // 4
        bb = jnp.where(is_ei, row % 4, row - 8)
        off = bb * (N + 1)
        dst = jnp.where(is_ei, r * (B * EC) + bb * EC, bb * EC) + k * CH

        @pl.when(is_ei & (k < KPR - 2))
        def _copy_add():
            pltpu.sync_copy(ei_hbm.at[pl.ds(r * E + k * CH, CH)], bufi)

            def addl(i, c):
                sl = pl.ds(i * 16, 16)
                bufi[sl] = bufi[sl] + off
                return c
            lax.fori_loop(0, VPC, addl, 0)
            pltpu.sync_copy(bufi, oi_hbm.at[pl.ds(dst, CH)])

        @pl.when(is_ei & (k >= KPR - 2))
        def _gen_virtual():
            vbase = (k - (KPR - 2)) * CH

            def genl(i, c):
                pos = lax.iota(jnp.int32, 16) + (vbase + i * 16)
                v0 = jnp.where(pos < N, N, pos - N)
                v1 = jnp.where(pos < N, pos, N)
                bufi[pl.ds(i * 16, 16)] = jnp.where(r == 0, v0, v1) + off
                return c
            lax.fori_loop(0, VPC, genl, 0)
            pltpu.sync_copy(bufi, oi_hbm.at[pl.ds(dst, CH)])

        @pl.when(jnp.logical_not(is_ei) & (k < KPR - 2))
        def _attr_copy():
            pltpu.sync_copy(ea_hbm.at[pl.ds(k * CH, CH)], buff)
            pltpu.sync_copy(buff, oa_hbm.at[pl.ds(dst, CH)])

        @pl.when(jnp.logical_not(is_ei) & (k >= KPR - 2))
        def _attr_ones():
            pltpu.sync_copy(onesv, oa_hbm.at[pl.ds(dst, CH)])

    def tloop(t, c):
        j = wid + NWORK * t

        @pl.when(j < NJOBS)
        def _():
            do_job(j)
        return c
    lax.fori_loop(0, (NJOBS + NWORK - 1) // NWORK, tloop, 0)

    @pl.when(wid >= NWORK - B)
    def _batch_vec():
        bb = wid - (NWORK - B)
        start = (bb * (N + 1)) // 8 * 8

        def bvl(i, c):
            pos = lax.iota(jnp.int32, 16) + (start + i * 16)
            bufbv[pl.ds(i * 16, 16)] = jnp.where(pos < bb * (N + 1), bb - 1, bb)
            return c
        lax.fori_loop(0, (N + 32) // 16, bvl, 0)

        @pl.when(bb < B - 1)
        def _():
            pltpu.sync_copy(bufbv.at[pl.ds(0, N)], ob_hbm.at[pl.ds(start, N)])

        @pl.when(bb == B - 1)
        def _():
            tail = NV - ((B - 1) * (N + 1)) // 8 * 8
            pltpu.sync_copy(bufbv.at[pl.ds(0, tail)], ob_hbm.at[pl.ds(start, tail)])


def kernel(obs, edge_index, edge_attr):
    idt = edge_index.dtype

    x = pl.pallas_call(
        _x_body,
        grid=(B,),
        in_specs=[pl.BlockSpec((1, N, D), lambda b: (b, 0, 0))],
        out_specs=pl.BlockSpec((NV, D), lambda b: (0, 0)),
        out_shape=jax.ShapeDtypeStruct((NV, D), obs.dtype),
    )(obs)

    edge_sc = functools.partial(
        pl.kernel,
        mesh=plsc.VectorSubcoreMesh(core_axis_name="c", subcore_axis_name="s"),
        out_type=[
            jax.ShapeDtypeStruct((2 * B * EC,), idt),
            jax.ShapeDtypeStruct((B * EC,), edge_attr.dtype),
            jax.ShapeDtypeStruct((NV,), jnp.int32),
        ],
        scratch_types=[
            pltpu.VMEM((CH,), jnp.int32),
            pltpu.VMEM((CH,), jnp.float32),
            pltpu.VMEM((CH,), jnp.float32),
            pltpu.VMEM((N + 48,), jnp.int32),
        ],
    )(_edge_sc_body)
    ones_chunk = jnp.ones((CH,), edge_attr.dtype)
    eif, eaf, batch_vec = edge_sc(edge_index.reshape(2 * E), edge_attr.reshape(E), ones_chunk)

    eindex = eif.reshape(2, B * EC)
    eattr = eaf.reshape(B * EC, 1)
    return x, eindex, eattr, batch_vec


# speedup vs baseline: 1.9188x; 1.3425x over previous
import functools

import jax
import jax.numpy as jnp
from jax import lax
from jax.experimental import pallas as pl
from jax.experimental.pallas import tpu as pltpu
from jax.experimental.pallas import tpu_sc as plsc

B, N, D, E = 4, 10000, 128, 160000
EC = E + 2 * N
NV = B * (N + 1)
W = 48000
NBLK = B * EC // W
CH = 10000
KPR = EC // CH
NJOBS = B * KPR
NWORK = 32
VPC = CH // 16


def _x_body(obs_ref, x_ref):
    b = pl.program_id(0)
    o = obs_ref[0]
    m = jnp.mean(o, axis=0, keepdims=True)

    for bb in range(B):
        @pl.when(b == bb)
        def _(bb=bb):
            x_ref[pl.ds(bb * (N + 1), N), :] = o
            x_ref[pl.ds(bb * (N + 1) + N, 1), :] = m


def _edge_tc_body(ei_ref, oi_ref):
    k = pl.program_id(0)
    for kk in range(NBLK):
        @pl.when(k == kk)
        def _(kk=kk):
            segs = []
            p, end = kk * W, kk * W + W
            while p < end:
                b, q = divmod(p, EC)
                off = b * (N + 1)
                if q < E:
                    seglen = min(E - q, end - p)
                    segs.append(ei_ref[:, q:q + seglen] + off)
                else:
                    vq = q - E
                    seglen = min(EC - q, end - p)
                    pos = lax.broadcasted_iota(jnp.int32, (1, seglen), 1) + vq
                    r0 = jnp.where(pos < N, N, pos - N)
                    r1 = jnp.where(pos < N, pos, N)
                    segs.append(jnp.concatenate([r0, r1], axis=0) + off)
                p += seglen
            oi_ref[...] = segs[0] if len(segs) == 1 else jnp.concatenate(segs, axis=1)


def _edge_sc_body(ea_hbm, ones_hbm, oa_hbm, ob_hbm, buff, onesv, bufbv):
    wid = lax.axis_index("c") * 16 + lax.axis_index("s")
    pltpu.sync_copy(ones_hbm, onesv)

    def do_job(j):
        bb = j // KPR
        k = j % KPR
        dst = bb * EC + k * CH

        @pl.when(k < KPR - 2)
        def _attr_copy():
            pltpu.sync_copy(ea_hbm.at[pl.ds(k * CH, CH)], buff)
            pltpu.sync_copy(buff, oa_hbm.at[pl.ds(dst, CH)])

        @pl.when(k >= KPR - 2)
        def _attr_ones():
            pltpu.sync_copy(onesv, oa_hbm.at[pl.ds(dst, CH)])

    def tloop(t, c):
        j = wid + NWORK * t

        @pl.when(j < NJOBS)
        def _():
            do_job(j)
        return c
    lax.fori_loop(0, (NJOBS + NWORK - 1) // NWORK, tloop, 0)

    @pl.when(wid >= NWORK - B)
    def _batch_vec():
        bb = wid - (NWORK - B)
        start = (bb * (N + 1)) // 8 * 8

        def bvl(i, c):
            pos = lax.iota(jnp.int32, 16) + (start + i * 16)
            bufbv[pl.ds(i * 16, 16)] = jnp.where(pos < bb * (N + 1), bb - 1, bb)
            return c
        lax.fori_loop(0, (N + 32) // 16, bvl, 0)

        @pl.when(bb < B - 1)
        def _():
            pltpu.sync_copy(bufbv.at[pl.ds(0, N)], ob_hbm.at[pl.ds(start, N)])

        @pl.when(bb == B - 1)
        def _():
            tail = NV - ((B - 1) * (N + 1)) // 8 * 8
            pltpu.sync_copy(bufbv.at[pl.ds(0, tail)], ob_hbm.at[pl.ds(start, tail)])


def kernel(obs, edge_index, edge_attr):
    idt = edge_index.dtype

    x = pl.pallas_call(
        _x_body,
        grid=(B,),
        in_specs=[pl.BlockSpec((1, N, D), lambda b: (b, 0, 0))],
        out_specs=pl.BlockSpec((NV, D), lambda b: (0, 0)),
        out_shape=jax.ShapeDtypeStruct((NV, D), obs.dtype),
    )(obs)

    eindex = pl.pallas_call(
        _edge_tc_body,
        grid=(NBLK,),
        in_specs=[pl.BlockSpec((2, E), lambda k: (0, 0))],
        out_specs=pl.BlockSpec((2, W), lambda k: (0, k)),
        out_shape=jax.ShapeDtypeStruct((2, B * EC), idt),
    )(edge_index)

    edge_sc = functools.partial(
        pl.kernel,
        mesh=plsc.VectorSubcoreMesh(core_axis_name="c", subcore_axis_name="s"),
        out_type=[
            jax.ShapeDtypeStruct((B * EC,), edge_attr.dtype),
            jax.ShapeDtypeStruct((NV,), jnp.int32),
        ],
        scratch_types=[
            pltpu.VMEM((CH,), jnp.float32),
            pltpu.VMEM((CH,), jnp.float32),
            pltpu.VMEM((N + 48,), jnp.int32),
        ],
    )(_edge_sc_body)
    ones_chunk = jnp.ones((CH,), edge_attr.dtype)
    eaf, batch_vec = edge_sc(edge_attr.reshape(E), ones_chunk)

    eattr = eaf.reshape(B * EC, 1)
    return x, eindex, eattr, batch_vec
